# baseline (speedup 1.0000x reference)
import jax
import jax.numpy as jnp
from jax import lax
from jax.experimental import pallas as pl

K = 8192
C = 32
N = 16384
TB = 256
NBLK = N // TB
BETA = 0.25


def _vq_block(x_ref, sx_ref, sw_ref, w_ref, out_ref, idx_ref, dmin_ref):
    xb = x_ref[...]
    w = w_ref[...]
    m = lax.dot_general(xb.astype(jnp.bfloat16), w.astype(jnp.bfloat16),
                        (((1,), (1,)), ((), ())),
                        preferred_element_type=jnp.float32)
    d2 = (sx_ref[...][:, None] - 2.0 * m) + sw_ref[...][None, :]
    dmin = jnp.min(d2, axis=1, keepdims=True)
    ids = lax.broadcasted_iota(jnp.int32, (TB, K), 1)
    idx = jnp.min(jnp.where(d2 == dmin, ids, K), axis=1)
    onehot = (ids == idx[:, None]).astype(jnp.float32)
    q = lax.dot_general(onehot, w, (((1,), (0,)), ((), ())),
                        precision=lax.Precision.HIGHEST,
                        preferred_element_type=jnp.float32)
    out_ref[...] = xb + (q - xb)
    idx_ref[...] = idx
    dmin_ref[...] = dmin


def kernel(x, W):
    xp = jnp.transpose(x, (0, 2, 3, 1))
    x_flat = xp.reshape(-1, C)
    s_x = jnp.sum(x_flat ** 2, axis=1)
    s_w = jnp.sum(W ** 2, axis=1)
    xq_flat, _, dmin = pl.pallas_call(
        _vq_block,
        grid=(NBLK,),
        in_specs=[
            pl.BlockSpec((TB, C), lambda i: (i, 0)),
            pl.BlockSpec((TB,), lambda i: (i,)),
            pl.BlockSpec((K,), lambda i: (0,)),
            pl.BlockSpec((K, C), lambda i: (0, 0)),
        ],
        out_specs=[
            pl.BlockSpec((TB, C), lambda i: (i, 0)),
            pl.BlockSpec((TB,), lambda i: (i,)),
            pl.BlockSpec((TB, 1), lambda i: (i, 0)),
        ],
        out_shape=[
            jax.ShapeDtypeStruct((N, C), jnp.float32),
            jax.ShapeDtypeStruct((N,), jnp.int32),
            jax.ShapeDtypeStruct((N, 1), jnp.float32),
        ],
    )(x_flat, s_x, s_w, W)
    loss = ((1.0 + BETA) / (N * C)) * jnp.sum(dmin)
    x_q = jnp.transpose(xq_flat.reshape(xp.shape), (0, 3, 1, 2))
    return (x_q, loss)
